# baseline (speedup 1.0000x reference)
import functools
import jax
import jax.numpy as jnp
from jax import lax
from jax.experimental import pallas as pl
from jax.experimental.pallas import tpu as pltpu
from jax.experimental.pallas import tpu_sc as plsc

N = 10000
E = 320000
D = 128
H = 128

NC = 2
NS = 16
NW = NC * NS

CHUNK = 128
NCHT = E // CHUNK
NCH = 80
TAIL = NCHT - 31 * NCH
TAILG = 16
TAILR = TAIL - TAILG
NPAD = 10240
RPT = NPAD // NS
G = 16
HC = 32
HPG = G * CHUNK // HC
EWK = NCH * CHUNK

_sc_params = pltpu.CompilerParams(needs_layout_passes=False, use_tc_tiling_on_sc=False)


@functools.cache
def _sc_kernels():
  mesh = plsc.VectorSubcoreMesh(core_axis_name="c", subcore_axis_name="s",
                                num_cores=NC, num_subcores=NS)

  @functools.partial(
      pl.kernel,
      out_type=jax.ShapeDtypeStruct((6 * NPAD,), jnp.float32),
      mesh=mesh,
      compiler_params=_sc_params,
      scratch_types=[
          pltpu.VMEM_SHARED((NPAD,), jnp.float32),
          pltpu.VMEM_SHARED((NPAD,), jnp.float32),
          pltpu.VMEM_SHARED((NPAD,), jnp.float32),
          pltpu.VMEM((NCH, CHUNK), jnp.int32),
          pltpu.VMEM((NCH, CHUNK), jnp.float32),
          pltpu.VMEM((CHUNK,), jnp.float32),
          pltpu.VMEM((RPT,), jnp.float32),
      ],
  )
  def _deg_kernel(cols0, w0, cols1, w1, cols2, w2, degp,
                  d0, d1, d2, cols_a, w_a, zbuf, vbuf):
    c = lax.axis_index("c")
    s = lax.axis_index("s")
    wid = s * NC + c
    z = jnp.zeros((16,), jnp.float32)
    for j in range(CHUNK // 16):
        zbuf[pl.ds(j * 16, 16)] = z
    for dk in (d0, d1, d2):
        for j in range(RPT // CHUNK):
            pltpu.sync_copy(zbuf, dk.at[pl.ds(s * RPT + j * CHUNK, CHUNK)])
    plsc.subcore_barrier()
    nch_w = jnp.where(wid == NW - 1, TAIL, NCH)
    for cols, w, dk in ((cols0, w0, d0), (cols1, w1, d1), (cols2, w2, d2)):
        @pl.when(wid < NW - 1)
        def _stage_full(cols=cols, w=w):
            pltpu.sync_copy(cols.at[pl.ds(wid * NCH, NCH)], cols_a)
            pltpu.sync_copy(w.at[pl.ds(wid * NCH, NCH)], w_a)

        @pl.when(wid == NW - 1)
        def _stage_tail(cols=cols, w=w):
            pltpu.sync_copy(cols.at[pl.ds((NW - 1) * NCH, TAIL)],
                            cols_a.at[pl.ds(0, TAIL)])
            pltpu.sync_copy(w.at[pl.ds((NW - 1) * NCH, TAIL)],
                            w_a.at[pl.ds(0, TAIL)])

        def chunk_body(ci, _, dk=dk):
            pltpu.sync_copy(w_a.at[ci], dk.at[cols_a.at[ci]], add=True)
            return ()
        lax.fori_loop(0, nch_w, chunk_body, ())
    plsc.subcore_barrier()
    for k, dk in enumerate((d0, d1, d2)):
        pltpu.sync_copy(dk.at[pl.ds(s * RPT, RPT)], vbuf)
        pltpu.sync_copy(vbuf, degp.at[pl.ds((c * 3 + k) * NPAD + s * RPT, RPT)])

  @functools.partial(
      pl.kernel,
      out_type=jax.ShapeDtypeStruct((3 * NW * NCH, CHUNK), jnp.float32),
      mesh=mesh,
      compiler_params=_sc_params,
      scratch_types=[
          pltpu.VMEM((NPAD,), jnp.float32),
          pltpu.VMEM((NCH, CHUNK), jnp.int32),
          pltpu.VMEM((NCH, CHUNK), jnp.int32),
          pltpu.VMEM((NCH, CHUNK), jnp.float32),
          pltpu.VMEM((NCH, CHUNK), jnp.float32),
      ],
  )
  def _norm_kernel(dis, r0, c0, w0, r1, c1, w1, r2, c2, w2,
                   normh, dis_v, rows_a, cols_a, w_a, norm_a):
    c = lax.axis_index("c")
    s = lax.axis_index("s")
    wid = s * NC + c
    nch_w = jnp.where(wid == NW - 1, TAIL, NCH)
    for k, (rows, cols, w) in enumerate(((r0, c0, w0), (r1, c1, w1), (r2, c2, w2))):
        pltpu.sync_copy(dis.at[pl.ds(k * NPAD, NPAD)], dis_v)

        @pl.when(wid < NW - 1)
        def _stage_full(rows=rows, cols=cols, w=w):
            pltpu.sync_copy(rows.at[pl.ds(wid * NCH, NCH)], rows_a)
            pltpu.sync_copy(cols.at[pl.ds(wid * NCH, NCH)], cols_a)
            pltpu.sync_copy(w.at[pl.ds(wid * NCH, NCH)], w_a)

        @pl.when(wid == NW - 1)
        def _stage_tail(rows=rows, cols=cols, w=w):
            pltpu.sync_copy(rows.at[pl.ds((NW - 1) * NCH, TAIL)],
                            rows_a.at[pl.ds(0, TAIL)])
            pltpu.sync_copy(cols.at[pl.ds((NW - 1) * NCH, TAIL)],
                            cols_a.at[pl.ds(0, TAIL)])
            pltpu.sync_copy(w.at[pl.ds((NW - 1) * NCH, TAIL)],
                            w_a.at[pl.ds(0, TAIL)])

        @plsc.parallel_loop(0, nch_w, unroll=2)
        def norm_body(j):
            for g in range(CHUNK // 16):
                r16 = rows_a[j, pl.ds(g * 16, 16)]
                c16 = cols_a[j, pl.ds(g * 16, 16)]
                dr = plsc.load_gather(dis_v, [r16])
                dc = plsc.load_gather(dis_v, [c16])
                norm_a[j, pl.ds(g * 16, 16)] = dr * w_a[j, pl.ds(g * 16, 16)] * dc

        @pl.when(wid < NW - 1)
        def _write_full(k=k):
            pltpu.sync_copy(norm_a, normh.at[pl.ds((k * NW + wid) * NCH, NCH)])

        @pl.when(wid == NW - 1)
        def _write_tail(k=k):
            pltpu.sync_copy(norm_a.at[pl.ds(0, TAIL)],
                            normh.at[pl.ds((k * NW + NW - 1) * NCH, TAIL)])

  @functools.partial(
      pl.kernel,
      out_type=jax.ShapeDtypeStruct((2, NPAD, H), jnp.float32),
      mesh=mesh,
      compiler_params=_sc_params,
      scratch_types=[
          pltpu.VMEM_SHARED((NPAD, H), jnp.float32),
          pltpu.VMEM((HC, H), jnp.bfloat16),
          pltpu.VMEM((HC, H), jnp.bfloat16),
          pltpu.VMEM((HC, H), jnp.bfloat16),
          pltpu.VMEM((HC, H), jnp.bfloat16),
          pltpu.VMEM((HC, H), jnp.float32),
          pltpu.VMEM((HC, H), jnp.float32),
          pltpu.VMEM((HC, H), jnp.float32),
          pltpu.VMEM((HC, H), jnp.float32),
          pltpu.VMEM((G * CHUNK,), jnp.int32),
          pltpu.VMEM((HPG, HC), jnp.int32),
          pltpu.VMEM((G * CHUNK,), jnp.float32),
          pltpu.SemaphoreType.DMA,
          pltpu.SemaphoreType.DMA,
          pltpu.SemaphoreType.DMA,
          pltpu.SemaphoreType.DMA,
          pltpu.SemaphoreType.DMA,
          pltpu.SemaphoreType.DMA,
          pltpu.SemaphoreType.DMA,
          pltpu.SemaphoreType.DMA,
      ],
  )
  def _agg_kernel(xw, normf, r0, c0, r1, c1, r2, c2,
                  out, acc, gb0, gb1, gb2, gb3, b0, b1, b2, b3,
                  rows_a, cols_a, norm_a,
                  g0, g1, g2, g3, s0, s1, s2, s3):
    c = lax.axis_index("c")
    s = lax.axis_index("s")
    wid = s * NC + c
    gbufs = (gb0, gb1, gb2, gb3)
    bufs = (b0, b1, b2, b3)
    gsems = (g0, g1, g2, g3)
    ssems = (s0, s1, s2, s3)
    z = jnp.zeros((16,), jnp.float32)

    @plsc.parallel_loop(0, HC, unroll=4)
    def _zero(i):
        for j in range(H // 16):
            b0[i, pl.ds(j * 16, 16)] = z
    for j in range(RPT // HC):
        pltpu.sync_copy(b0, acc.at[pl.ds(s * RPT + j * HC, HC)])
    plsc.subcore_barrier()

    def scale(gbuf, buf, hc):
        base = jnp.zeros((16,), jnp.int32) + hc * HC

        @plsc.parallel_loop(0, HC, unroll=4)
        def scale_body(e):
            nb = plsc.load_gather(norm_a, [base + e])
            for j in range(H // 32):
                v = gbuf[e, pl.ds(j * 32, 32)]
                va, vb = plsc.unpack(v, format=plsc.PackFormat.INTERLEAVED)
                buf[e, pl.ds(j * 32, 16)] = va * nb
                buf[e, pl.ds(j * 32 + 16, 16)] = vb * nb

    def gidx(hc):
        return rows_a.at[pl.ds(hc * HC, HC)]

    def run_group(rowsf, colsh, ebase, hbase, nbase, nhc):
        ne = nhc * HC
        pltpu.sync_copy(rowsf.at[pl.ds(ebase, ne)], rows_a.at[pl.ds(0, ne)])
        pltpu.sync_copy(colsh.at[pl.ds(hbase, nhc)], cols_a.at[pl.ds(0, nhc)])
        pltpu.sync_copy(normf.at[pl.ds(nbase, ne)], norm_a.at[pl.ds(0, ne)])
        pltpu.async_copy(xw.at[gidx(0)], gb0, g0)
        pltpu.async_copy(xw.at[gidx(1)], gb1, g1)
        pltpu.async_copy(xw.at[gidx(2)], gb2, g2)
        pltpu.async_copy(xw.at[gidx(3)], gb3, g3)

        def quad(i, _):
            for b in range(4):
                hc = 4 * i + b
                gbuf, buf, gs = gbufs[b], bufs[b], gsems[b]
                bb2 = (b + 2) % 4
                pltpu.make_async_copy(xw.at[gidx(hc)], gbuf, gs).wait()
                scale(gbuf, buf, hc)
                pltpu.async_copy(buf, acc.at[cols_a.at[hc]], ssems[b], add=True)
                def maint(hc=hc, bb2=bb2):
                    pltpu.make_async_copy(
                        bufs[bb2], acc.at[cols_a.at[hc - 2]], ssems[bb2]).wait()
                    pltpu.async_copy(xw.at[gidx(hc + 2)], gbufs[bb2], gsems[bb2])
                if b < 2:
                    pl.when(i > 0)(maint)
                else:
                    pl.when(i < nhc // 4 - 1)(maint)
            return ()
        lax.fori_loop(0, nhc // 4, quad, ())
        for t in range(4):
            hc = nhc - 4 + t
            pltpu.make_async_copy(bufs[hc % 4], acc.at[cols_a.at[hc]],
                                  ssems[hc % 4]).wait()

    ngroups = jnp.where(wid == NW - 1, TAILG // G, NCH // G)
    for k, (rowsf, colsh) in enumerate(((r0, c0), (r1, c1), (r2, c2))):
        def group_body(g, _, rowsf=rowsf, colsh=colsh, k=k):
            run_group(rowsf, colsh,
                      wid * EWK + g * G * CHUNK,
                      (wid * EWK + g * G * CHUNK) // HC,
                      (k * NW + wid) * EWK + g * G * CHUNK,
                      HPG)
            return ()
        lax.fori_loop(0, ngroups, group_body, ())

        @pl.when(wid == NW - 1)
        def _tail(rowsf=rowsf, colsh=colsh, k=k):
            loff = TAILG * CHUNK
            run_group(rowsf, colsh,
                      (NW - 1) * EWK + loff,
                      ((NW - 1) * EWK + loff) // HC,
                      (k * NW + NW - 1) * EWK + loff,
                      TAILR * CHUNK // HC)

    plsc.subcore_barrier()
    for j in range(RPT // CHUNK):
        pltpu.sync_copy(acc.at[pl.ds(s * RPT + j * CHUNK, CHUNK)],
                        out.at[c, pl.ds(s * RPT + j * CHUNK, CHUNK)])

  return _deg_kernel, _norm_kernel, _agg_kernel


def _mm_body(x_ref, w_ref, o_ref):
    o_ref[...] = jnp.dot(x_ref[...], w_ref[...],
                         preferred_element_type=jnp.float32).astype(jnp.bfloat16)


def _dis_body(degp_ref, dis_ref):
    deg = degp_ref[0:3, :] + degp_ref[3:6, :]
    safe = jnp.where(deg > 0, deg, 1.0)
    dis_ref[...] = jnp.where(deg > 0, lax.rsqrt(safe), 0.0)


def _final_body(p0_ref, p1_ref, b_ref, o_ref):
    s = p0_ref[0] + p1_ref[0] + 3.0 * b_ref[...]
    o_ref[...] = jnp.maximum(s, 0.0)


def _chunk_edges(ei, ew):
    return (ei[0].reshape(NCHT, CHUNK), ei[1].reshape(NCHT, CHUNK),
            ew.reshape(NCHT, CHUNK))


@jax.jit
def kernel(x, edge_index0, edge_weight0, edge_index1, edge_weight1,
           edge_index2, edge_weight2, W, b):
    r0, c0, w0 = _chunk_edges(edge_index0, edge_weight0)
    r1, c1, w1 = _chunk_edges(edge_index1, edge_weight1)
    r2, c2, w2 = _chunk_edges(edge_index2, edge_weight2)

    deg_kernel, norm_kernel, agg_kernel = _sc_kernels()
    degp = deg_kernel(c0, w0, c1, w1, c2, w2).reshape(6, NPAD)

    xw = pl.pallas_call(
        _mm_body,
        out_shape=jax.ShapeDtypeStruct((N, H), jnp.bfloat16),
        grid=(10,),
        in_specs=[pl.BlockSpec((1000, D), lambda i: (i, 0)),
                  pl.BlockSpec((D, H), lambda i: (0, 0))],
        out_specs=pl.BlockSpec((1000, H), lambda i: (i, 0)),
    )(x, W)

    dis = pl.pallas_call(
        _dis_body,
        out_shape=jax.ShapeDtypeStruct((3, NPAD), jnp.float32),
        in_specs=[pl.BlockSpec((6, NPAD), lambda: (0, 0))],
        out_specs=pl.BlockSpec((3, NPAD), lambda: (0, 0)),
    )(degp)

    normh = norm_kernel(dis.reshape(3 * NPAD), r0, c0, w0, r1, c1, w1, r2, c2, w2)
    p = agg_kernel(xw, normh.reshape(-1),
                   edge_index0[0], edge_index0[1].reshape(E // HC, HC),
                   edge_index1[0], edge_index1[1].reshape(E // HC, HC),
                   edge_index2[0], edge_index2[1].reshape(E // HC, HC))

    b_shuf = b.reshape(4, 16, 2).transpose(0, 2, 1).reshape(1, H)
    out = pl.pallas_call(
        _final_body,
        out_shape=jax.ShapeDtypeStruct((N, H), jnp.float32),
        grid=(10,),
        in_specs=[pl.BlockSpec((1, 1000, H), lambda i: (0, i, 0)),
                  pl.BlockSpec((1, 1000, H), lambda i: (1, i, 0)),
                  pl.BlockSpec((1, H), lambda i: (0, 0))],
        out_specs=pl.BlockSpec((1000, H), lambda i: (i, 0)),
    )(p, p, b_shuf)
    return out.reshape(N, 4, 2, 16).transpose(0, 1, 3, 2).reshape(N, H)


# speedup vs baseline: 1.1685x; 1.1685x over previous
import functools
import jax
import jax.numpy as jnp
from jax import lax
from jax.experimental import pallas as pl
from jax.experimental.pallas import tpu as pltpu
from jax.experimental.pallas import tpu_sc as plsc

N = 10000
E = 320000
D = 128
H = 128

NC = 2
NS = 16
NW = NC * NS

CHUNK = 128
NCHT = E // CHUNK
NCH = 80
TAIL = NCHT - 31 * NCH
TAILG = 0
TAILR = TAIL - TAILG
NPAD = 10240
RPT = NPAD // NS
G = 40
HC = 64
HPG = G * CHUNK // HC
EWK = NCH * CHUNK

_sc_params = pltpu.CompilerParams(needs_layout_passes=False, use_tc_tiling_on_sc=False)


@functools.cache
def _sc_kernels():
  mesh = plsc.VectorSubcoreMesh(core_axis_name="c", subcore_axis_name="s",
                                num_cores=NC, num_subcores=NS)

  @functools.partial(
      pl.kernel,
      out_type=jax.ShapeDtypeStruct((6 * NPAD,), jnp.float32),
      mesh=mesh,
      compiler_params=_sc_params,
      scratch_types=[
          pltpu.VMEM_SHARED((NPAD,), jnp.float32),
          pltpu.VMEM_SHARED((NPAD,), jnp.float32),
          pltpu.VMEM_SHARED((NPAD,), jnp.float32),
          pltpu.VMEM((NCH, CHUNK), jnp.int32),
          pltpu.VMEM((NCH, CHUNK), jnp.float32),
          pltpu.VMEM((CHUNK,), jnp.float32),
          pltpu.VMEM((RPT,), jnp.float32),
      ],
  )
  def _deg_kernel(cols0, w0, cols1, w1, cols2, w2, degp,
                  d0, d1, d2, cols_a, w_a, zbuf, vbuf):
    c = lax.axis_index("c")
    s = lax.axis_index("s")
    wid = s * NC + c
    z = jnp.zeros((16,), jnp.float32)
    for j in range(CHUNK // 16):
        zbuf[pl.ds(j * 16, 16)] = z
    for dk in (d0, d1, d2):
        for j in range(RPT // CHUNK):
            pltpu.sync_copy(zbuf, dk.at[pl.ds(s * RPT + j * CHUNK, CHUNK)])
    plsc.subcore_barrier()
    nch_w = jnp.where(wid == NW - 1, TAIL, NCH)
    for cols, w, dk in ((cols0, w0, d0), (cols1, w1, d1), (cols2, w2, d2)):
        @pl.when(wid < NW - 1)
        def _stage_full(cols=cols, w=w):
            pltpu.sync_copy(cols.at[pl.ds(wid * NCH, NCH)], cols_a)
            pltpu.sync_copy(w.at[pl.ds(wid * NCH, NCH)], w_a)

        @pl.when(wid == NW - 1)
        def _stage_tail(cols=cols, w=w):
            pltpu.sync_copy(cols.at[pl.ds((NW - 1) * NCH, TAIL)],
                            cols_a.at[pl.ds(0, TAIL)])
            pltpu.sync_copy(w.at[pl.ds((NW - 1) * NCH, TAIL)],
                            w_a.at[pl.ds(0, TAIL)])

        def chunk_body(ci, _, dk=dk):
            pltpu.sync_copy(w_a.at[ci], dk.at[cols_a.at[ci]], add=True)
            return ()
        lax.fori_loop(0, nch_w, chunk_body, ())
    plsc.subcore_barrier()
    for k, dk in enumerate((d0, d1, d2)):
        pltpu.sync_copy(dk.at[pl.ds(s * RPT, RPT)], vbuf)
        pltpu.sync_copy(vbuf, degp.at[pl.ds((c * 3 + k) * NPAD + s * RPT, RPT)])

  @functools.partial(
      pl.kernel,
      out_type=jax.ShapeDtypeStruct((3 * NW * NCH, CHUNK), jnp.float32),
      mesh=mesh,
      compiler_params=_sc_params,
      scratch_types=[
          pltpu.VMEM((NPAD,), jnp.float32),
          pltpu.VMEM((NCH, CHUNK), jnp.int32),
          pltpu.VMEM((NCH, CHUNK), jnp.int32),
          pltpu.VMEM((NCH, CHUNK), jnp.float32),
          pltpu.VMEM((NCH, CHUNK), jnp.float32),
      ],
  )
  def _norm_kernel(dis, r0, c0, w0, r1, c1, w1, r2, c2, w2,
                   normh, dis_v, rows_a, cols_a, w_a, norm_a):
    c = lax.axis_index("c")
    s = lax.axis_index("s")
    wid = s * NC + c
    nch_w = jnp.where(wid == NW - 1, TAIL, NCH)
    for k, (rows, cols, w) in enumerate(((r0, c0, w0), (r1, c1, w1), (r2, c2, w2))):
        pltpu.sync_copy(dis.at[pl.ds(k * NPAD, NPAD)], dis_v)

        @pl.when(wid < NW - 1)
        def _stage_full(rows=rows, cols=cols, w=w):
            pltpu.sync_copy(rows.at[pl.ds(wid * NCH, NCH)], rows_a)
            pltpu.sync_copy(cols.at[pl.ds(wid * NCH, NCH)], cols_a)
            pltpu.sync_copy(w.at[pl.ds(wid * NCH, NCH)], w_a)

        @pl.when(wid == NW - 1)
        def _stage_tail(rows=rows, cols=cols, w=w):
            pltpu.sync_copy(rows.at[pl.ds((NW - 1) * NCH, TAIL)],
                            rows_a.at[pl.ds(0, TAIL)])
            pltpu.sync_copy(cols.at[pl.ds((NW - 1) * NCH, TAIL)],
                            cols_a.at[pl.ds(0, TAIL)])
            pltpu.sync_copy(w.at[pl.ds((NW - 1) * NCH, TAIL)],
                            w_a.at[pl.ds(0, TAIL)])

        @plsc.parallel_loop(0, nch_w, unroll=2)
        def norm_body(j):
            for g in range(CHUNK // 16):
                r16 = rows_a[j, pl.ds(g * 16, 16)]
                c16 = cols_a[j, pl.ds(g * 16, 16)]
                dr = plsc.load_gather(dis_v, [r16])
                dc = plsc.load_gather(dis_v, [c16])
                norm_a[j, pl.ds(g * 16, 16)] = dr * w_a[j, pl.ds(g * 16, 16)] * dc

        @pl.when(wid < NW - 1)
        def _write_full(k=k):
            pltpu.sync_copy(norm_a, normh.at[pl.ds((k * NW + wid) * NCH, NCH)])

        @pl.when(wid == NW - 1)
        def _write_tail(k=k):
            pltpu.sync_copy(norm_a.at[pl.ds(0, TAIL)],
                            normh.at[pl.ds((k * NW + NW - 1) * NCH, TAIL)])

  @functools.partial(
      pl.kernel,
      out_type=jax.ShapeDtypeStruct((2, NPAD, H), jnp.float32),
      mesh=mesh,
      compiler_params=_sc_params,
      scratch_types=[
          pltpu.VMEM_SHARED((NPAD, H), jnp.float32),
          pltpu.VMEM((HC, H), jnp.float32),
          pltpu.VMEM((HC, H), jnp.float32),
          pltpu.VMEM((HC, H), jnp.float32),
          pltpu.VMEM((HC, H), jnp.float32),
          pltpu.VMEM((G * CHUNK,), jnp.int32),
          pltpu.VMEM((HPG, HC), jnp.int32),
          pltpu.VMEM((G * CHUNK,), jnp.float32),
          pltpu.SemaphoreType.DMA,
          pltpu.SemaphoreType.DMA,
          pltpu.SemaphoreType.DMA,
          pltpu.SemaphoreType.DMA,
          pltpu.SemaphoreType.DMA,
          pltpu.SemaphoreType.DMA,
          pltpu.SemaphoreType.DMA,
          pltpu.SemaphoreType.DMA,
      ],
  )
  def _agg_kernel(xw, normf, r0, c0, r1, c1, r2, c2,
                  out, acc, b0, b1, b2, b3, rows_a, cols_a, norm_a,
                  g0, g1, g2, g3, s0, s1, s2, s3):
    c = lax.axis_index("c")
    s = lax.axis_index("s")
    wid = s * NC + c
    bufs = (b0, b1, b2, b3)
    gsems = (g0, g1, g2, g3)
    ssems = (s0, s1, s2, s3)
    z = jnp.zeros((16,), jnp.float32)

    @plsc.parallel_loop(0, HC, unroll=4)
    def _zero(i):
        for j in range(H // 16):
            b0[i, pl.ds(j * 16, 16)] = z
    for j in range(RPT // HC):
        pltpu.sync_copy(b0, acc.at[pl.ds(s * RPT + j * HC, HC)])
    plsc.subcore_barrier()

    def scale(buf, hc):
        base = jnp.zeros((16,), jnp.int32) + hc * HC

        @plsc.parallel_loop(0, HC, unroll=4)
        def scale_body(e):
            nb = plsc.load_gather(norm_a, [base + e])
            for j in range(H // 16):
                buf[e, pl.ds(j * 16, 16)] = buf[e, pl.ds(j * 16, 16)] * nb

    def gidx(hc):
        return rows_a.at[pl.ds(hc * HC, HC)]

    def run_group(rowsf, colsh, ebase, hbase, nbase, nhc):
        ne = nhc * HC
        pltpu.sync_copy(rowsf.at[pl.ds(ebase, ne)], rows_a.at[pl.ds(0, ne)])
        pltpu.sync_copy(colsh.at[pl.ds(hbase, nhc)], cols_a.at[pl.ds(0, nhc)])
        pltpu.sync_copy(normf.at[pl.ds(nbase, ne)], norm_a.at[pl.ds(0, ne)])
        pltpu.async_copy(xw.at[gidx(0)], b0, g0)
        pltpu.async_copy(xw.at[gidx(1)], b1, g1)
        pltpu.async_copy(xw.at[gidx(2)], b2, g2)
        pltpu.async_copy(xw.at[gidx(3)], b3, g3)

        def quad(i, _):
            for b in range(4):
                hc = 4 * i + b
                buf, gs = bufs[b], gsems[b]
                bb2 = (b + 2) % 4
                pltpu.make_async_copy(xw.at[gidx(hc)], buf, gs).wait()
                scale(buf, hc)
                pltpu.async_copy(buf, acc.at[cols_a.at[hc]], ssems[b], add=True)
                def maint(hc=hc, bb2=bb2):
                    pltpu.make_async_copy(
                        bufs[bb2], acc.at[cols_a.at[hc - 2]], ssems[bb2]).wait()
                    pltpu.async_copy(xw.at[gidx(hc + 2)], bufs[bb2], gsems[bb2])
                if b < 2:
                    pl.when(i > 0)(maint)
                else:
                    pl.when(i < nhc // 4 - 1)(maint)
            return ()
        lax.fori_loop(0, nhc // 4, quad, ())
        for t in range(4):
            hc = nhc - 4 + t
            pltpu.make_async_copy(bufs[hc % 4], acc.at[cols_a.at[hc]],
                                  ssems[hc % 4]).wait()

    ngroups = jnp.where(wid == NW - 1, TAILG // G, NCH // G)
    for k, (rowsf, colsh) in enumerate(((r0, c0), (r1, c1), (r2, c2))):
        def group_body(g, _, rowsf=rowsf, colsh=colsh, k=k):
            run_group(rowsf, colsh,
                      wid * EWK + g * G * CHUNK,
                      (wid * EWK + g * G * CHUNK) // HC,
                      (k * NW + wid) * EWK + g * G * CHUNK,
                      HPG)
            return ()
        lax.fori_loop(0, ngroups, group_body, ())

        @pl.when(wid == NW - 1)
        def _tail(rowsf=rowsf, colsh=colsh, k=k):
            loff = TAILG * CHUNK
            run_group(rowsf, colsh,
                      (NW - 1) * EWK + loff,
                      ((NW - 1) * EWK + loff) // HC,
                      (k * NW + NW - 1) * EWK + loff,
                      TAILR * CHUNK // HC)

    plsc.subcore_barrier()
    for j in range(RPT // CHUNK):
        pltpu.sync_copy(acc.at[pl.ds(s * RPT + j * CHUNK, CHUNK)],
                        out.at[c, pl.ds(s * RPT + j * CHUNK, CHUNK)])

  return _deg_kernel, _norm_kernel, _agg_kernel


def _mm_body(x_ref, w_ref, o_ref):
    o_ref[...] = jnp.dot(x_ref[...], w_ref[...],
                         preferred_element_type=jnp.float32)


def _dis_body(degp_ref, dis_ref):
    deg = degp_ref[0:3, :] + degp_ref[3:6, :]
    safe = jnp.where(deg > 0, deg, 1.0)
    dis_ref[...] = jnp.where(deg > 0, lax.rsqrt(safe), 0.0)


def _final_body(p0_ref, p1_ref, b_ref, o_ref):
    s = p0_ref[0] + p1_ref[0] + 3.0 * b_ref[...]
    o_ref[...] = jnp.maximum(s, 0.0)


def _chunk_edges(ei, ew):
    return (ei[0].reshape(NCHT, CHUNK), ei[1].reshape(NCHT, CHUNK),
            ew.reshape(NCHT, CHUNK))


@jax.jit
def kernel(x, edge_index0, edge_weight0, edge_index1, edge_weight1,
           edge_index2, edge_weight2, W, b):
    r0, c0, w0 = _chunk_edges(edge_index0, edge_weight0)
    r1, c1, w1 = _chunk_edges(edge_index1, edge_weight1)
    r2, c2, w2 = _chunk_edges(edge_index2, edge_weight2)

    deg_kernel, norm_kernel, agg_kernel = _sc_kernels()
    degp = deg_kernel(c0, w0, c1, w1, c2, w2).reshape(6, NPAD)

    xw = pl.pallas_call(
        _mm_body,
        out_shape=jax.ShapeDtypeStruct((N, H), jnp.float32),
        grid=(10,),
        in_specs=[pl.BlockSpec((1000, D), lambda i: (i, 0)),
                  pl.BlockSpec((D, H), lambda i: (0, 0))],
        out_specs=pl.BlockSpec((1000, H), lambda i: (i, 0)),
    )(x, W)

    dis = pl.pallas_call(
        _dis_body,
        out_shape=jax.ShapeDtypeStruct((3, NPAD), jnp.float32),
        in_specs=[pl.BlockSpec((6, NPAD), lambda: (0, 0))],
        out_specs=pl.BlockSpec((3, NPAD), lambda: (0, 0)),
    )(degp)

    normh = norm_kernel(dis.reshape(3 * NPAD), r0, c0, w0, r1, c1, w1, r2, c2, w2)
    p = agg_kernel(xw, normh.reshape(-1),
                   edge_index0[0], edge_index0[1].reshape(E // HC, HC),
                   edge_index1[0], edge_index1[1].reshape(E // HC, HC),
                   edge_index2[0], edge_index2[1].reshape(E // HC, HC))

    b2 = b.reshape(1, H)
    out = pl.pallas_call(
        _final_body,
        out_shape=jax.ShapeDtypeStruct((N, H), jnp.float32),
        grid=(10,),
        in_specs=[pl.BlockSpec((1, 1000, H), lambda i: (0, i, 0)),
                  pl.BlockSpec((1, 1000, H), lambda i: (1, i, 0)),
                  pl.BlockSpec((1, H), lambda i: (0, 0))],
        out_specs=pl.BlockSpec((1000, H), lambda i: (i, 0)),
    )(p, p, b2)
    return out
